# baseline (speedup 1.0000x reference)
import functools

import jax
import jax.numpy as jnp
from jax import lax
from jax.experimental import pallas as pl
from jax.experimental.pallas import tpu as pltpu
from jax.experimental.pallas import tpu_sc as plsc

BATCH = 4096
SEQ = 200
EMBED_DIM = 64
HIDDEN = 256
NUM_CLASSES = 50

NC = 2
NS = 16
NW = NC * NS
BPW = BATCH // NW
CHUNK_A = 104
CHUNK_B = SEQ - CHUNK_A
LANES = 16
NCOL = EMBED_DIM // LANES


def _pool_body(x_hbm, emb_hbm, out_hbm, idx_v, rows_v, pooled_v, sem0, sem1):
    wid = lax.axis_index("s") * NC + lax.axis_index("c")
    base = wid * BPW
    pltpu.sync_copy(x_hbm.at[pl.ds(base * SEQ, BPW * SEQ)], idx_v)

    sems = (sem0, sem1)

    def issue(b, slot):
        sem = sems[slot]
        pltpu.async_copy(
            emb_hbm.at[idx_v.at[pl.ds(b * SEQ, CHUNK_A)]],
            rows_v.at[pl.ds(slot * SEQ, CHUNK_A)],
            sem,
        )
        pltpu.async_copy(
            emb_hbm.at[idx_v.at[pl.ds(b * SEQ + CHUNK_A, CHUNK_B)]],
            rows_v.at[pl.ds(slot * SEQ + CHUNK_A, CHUNK_B)],
            sem,
        )

    def drain(slot):
        pltpu.make_async_copy(
            emb_hbm.at[pl.ds(0, SEQ)],
            rows_v.at[pl.ds(slot * SEQ, SEQ)],
            sems[slot],
        ).wait()

    inv = jnp.float32(1.0 / SEQ)

    def reduce_item(b, slot):
        def row_step(r, acc):
            return tuple(
                acc[c] + rows_v[slot * SEQ + r, pl.ds(c * LANES, LANES)]
                for c in range(NCOL)
            )

        acc0 = tuple(jnp.zeros((LANES,), jnp.float32) for _ in range(NCOL))
        acc = lax.fori_loop(0, SEQ, row_step, acc0, unroll=4)
        for c in range(NCOL):
            pooled_v[pl.ds(b * EMBED_DIM + c * LANES, LANES)] = acc[c] * inv

    issue(0, 0)
    issue(1, 1)

    def outer(i, _):
        g = i * 2
        for s in range(2):
            b = g + s
            drain(s)
            reduce_item(b, s)
            issue(b + 2, s)
        return 0

    lax.fori_loop(0, (BPW - 2) // 2, outer, 0)
    for s in range(2):
        drain(s)
        reduce_item(BPW - 2 + s, s)

    pltpu.sync_copy(
        pooled_v, out_hbm.at[pl.ds(base * EMBED_DIM, BPW * EMBED_DIM)]
    )


_pool = functools.partial(
    pl.kernel,
    out_type=jax.ShapeDtypeStruct((BATCH * EMBED_DIM,), jnp.float32),
    mesh=plsc.VectorSubcoreMesh(core_axis_name="c", subcore_axis_name="s"),
    compiler_params=pltpu.CompilerParams(use_tc_tiling_on_sc=False),
    scratch_types=[
        pltpu.VMEM((BPW * SEQ,), jnp.int32),
        pltpu.VMEM((2 * SEQ, EMBED_DIM), jnp.float32),
        pltpu.VMEM((BPW * EMBED_DIM,), jnp.float32),
        pltpu.SemaphoreType.DMA,
        pltpu.SemaphoreType.DMA,
    ],
)(_pool_body)


def _mlp_body(p_ref, w1_ref, b1_ref, w2_ref, b2_ref, o_ref):
    h = jnp.dot(p_ref[...], w1_ref[...], preferred_element_type=jnp.float32)
    h = jnp.maximum(h + b1_ref[...], 0.0)
    o_ref[...] = (
        jnp.dot(h, w2_ref[...], preferred_element_type=jnp.float32)
        + b2_ref[...]
    )


_MLP_BLOCK = 512


@jax.jit
def kernel(x, emb, W1, b1, W2, b2):
    x = x.astype(jnp.int32).reshape(BATCH * SEQ)
    pooled = _pool(x, emb).reshape(BATCH, EMBED_DIM)
    grid = BATCH // _MLP_BLOCK
    out = pl.pallas_call(
        _mlp_body,
        grid=(grid,),
        in_specs=[
            pl.BlockSpec((_MLP_BLOCK, EMBED_DIM), lambda i: (i, 0)),
            pl.BlockSpec((EMBED_DIM, HIDDEN), lambda i: (0, 0)),
            pl.BlockSpec((1, HIDDEN), lambda i: (0, 0)),
            pl.BlockSpec((HIDDEN, NUM_CLASSES), lambda i: (0, 0)),
            pl.BlockSpec((1, NUM_CLASSES), lambda i: (0, 0)),
        ],
        out_specs=pl.BlockSpec((_MLP_BLOCK, NUM_CLASSES), lambda i: (i, 0)),
        out_shape=jax.ShapeDtypeStruct((BATCH, NUM_CLASSES), jnp.float32),
    )(pooled, W1, b1.reshape(1, HIDDEN), W2, b2.reshape(1, NUM_CLASSES))
    return out
